# baseline (speedup 1.0000x reference)
import functools

import jax
import jax.numpy as jnp
from jax import lax
from jax.experimental import pallas as pl
from jax.experimental.pallas import tpu as pltpu
from jax.experimental.pallas import tpu_sc as plsc

S = 32768
E = 16
B = 32
L = 16
G = 128
RPB = S // G
CHUNK = 16
ROWS = RPB + CHUNK
NT = 16
SRCROWS = E * S // G


def _worker(x_hbm, idx_hbm, out_hbm, idx_v, zbuf, buf_a, buf_b, accsh,
            sem_a, sem_b, ssem_a, ssem_b, zsem):
    c = lax.axis_index("c")
    s = lax.axis_index("s")
    b = s * 2 + c

    pltpu.sync_copy(idx_hbm.at[b], idx_v)
    vec = idx_v[...]

    lanes = lax.iota(jnp.int32, L)
    zeros = jnp.zeros((L,), jnp.float32)

    bufs = (buf_a, buf_b)
    sems = (sem_a, sem_b)
    ssems = (ssem_a, ssem_b)
    starts = [vec[i] * 2 for i in range(E)]
    nblocks = [(RPB - starts[i] + CHUNK - 1) // CHUNK for i in range(E)]

    acc = accsh.at[s]

    for r in range(CHUNK):
        for cj in range(G // L):
            zbuf[r, pl.ds(cj * L, L)] = zeros

    def zero_rows(k, carry):
        pltpu.make_async_copy(zbuf, acc.at[pl.ds(k * CHUNK, CHUNK)],
                              zsem).start()
        return carry

    nzb = ROWS // CHUNK
    lax.fori_loop(0, nzb, zero_rows, None)

    def drain_zero(k, carry):
        pltpu.make_async_copy(zbuf, acc.at[pl.ds(k * CHUNK, CHUNK)],
                              zsem).wait()
        return carry

    lax.fori_loop(0, nzb, drain_zero, None)

    def stage(i, is_start):
        buf = bufs[i % 2]
        sem = sems[i % 2]
        src0 = RPB * i

        def body(k, carry):
            copy = pltpu.make_async_copy(
                x_hbm.at[b, pl.ds(src0 + k * CHUNK, CHUNK)],
                buf.at[pl.ds(k * CHUNK, CHUNK)],
                sem,
            )
            if is_start:
                copy.start()
            else:
                copy.wait()
            return carry

        lax.fori_loop(0, nblocks[i], body, None)

    def scatter(i, is_start):
        buf = bufs[i % 2]
        ssem = ssems[i % 2]
        st = starts[i]

        def body(k, carry):
            idx_vec = st + k * CHUNK + lanes
            src = buf.at[pl.ds(k * CHUNK, CHUNK)]
            if is_start:
                pltpu.async_copy(src, acc.at[idx_vec], ssem, add=True)
            else:
                pltpu.make_async_copy(src, acc.at[idx_vec], ssem).wait()
            return carry

        lax.fori_loop(0, nblocks[i], body, None)

    stage(0, True)
    for i in range(E):
        stage(i, False)
        scatter(i, True)
        if i + 1 < E:
            if i >= 1:
                scatter(i - 1, False)
            stage(i + 1, True)
    scatter(E - 2, False)
    scatter(E - 1, False)

    pltpu.sync_copy(acc.at[pl.ds(0, RPB)], out_hbm.at[b])


_mesh = plsc.VectorSubcoreMesh(core_axis_name="c", subcore_axis_name="s")

_summarize = functools.partial(
    pl.kernel,
    mesh=_mesh,
    out_type=jax.ShapeDtypeStruct((B, RPB, G), jnp.float32),
    scratch_types=[
        pltpu.VMEM((E,), jnp.int32),
        pltpu.VMEM((CHUNK, G), jnp.float32),
        pltpu.VMEM((RPB, G), jnp.float32),
        pltpu.VMEM((RPB, G), jnp.float32),
        pltpu.VMEM_SHARED((NT, ROWS, G), jnp.float32),
        pltpu.SemaphoreType.DMA,
        pltpu.SemaphoreType.DMA,
        pltpu.SemaphoreType.DMA,
        pltpu.SemaphoreType.DMA,
        pltpu.SemaphoreType.DMA,
    ],
)(_worker)


def kernel(x, indices):
    xr = x.reshape(B, SRCROWS, G)
    out = _summarize(xr, indices.astype(jnp.int32))
    return out.reshape(B, 1, S)


# speedup vs baseline: 2.2805x; 2.2805x over previous
import functools

import jax
import jax.numpy as jnp
from jax import lax
from jax.experimental import pallas as pl
from jax.experimental.pallas import tpu as pltpu
from jax.experimental.pallas import tpu_sc as plsc

S = 32768
E = 16
B = 32
STEP = 256
L = 16
CHUNK = 2048
TAILS = (1024, 512, 256)
UNROLL = 16


def _worker(x_hbm, idx_hbm, out_hbm, idx_v, acc, buf_a, buf_b, sem_a, sem_b):
    c = lax.axis_index("c")
    s = lax.axis_index("s")
    b = s * 2 + c

    pltpu.sync_copy(idx_hbm.at[b], idx_v)
    vec = idx_v[...]

    zeros = jnp.zeros((L,), jnp.float32)

    bufs = (buf_a, buf_b)
    sems = (sem_a, sem_b)
    starts = [vec[i] * STEP for i in range(E)]
    nwords = [S - starts[i] for i in range(E)]
    nfulls = [nwords[i] // CHUNK for i in range(E)]
    tails = [nwords[i] - nfulls[i] * CHUNK for i in range(E)]

    def stage(i, is_start):
        if i == 0:
            dst = lambda o, n: acc.at[pl.ds(starts[0] + o, n)]
        else:
            buf = bufs[i % 2]
            dst = lambda o, n: buf.at[pl.ds(o, n)]
        sem = sems[i % 2]

        def move(o, n):
            copy = pltpu.make_async_copy(
                x_hbm.at[b, i, pl.ds(o, n)], dst(o, n), sem)
            if is_start:
                copy.start()
            else:
                copy.wait()

        def body(k, carry):
            move(k * CHUNK, CHUNK)
            return carry

        lax.fori_loop(0, nfulls[i], body, None)
        off = nfulls[i] * CHUNK
        for sz in TAILS:
            @pl.when((tails[i] & sz) != 0)
            def _piece(o=pl.multiple_of(off, STEP), n=sz):
                move(o, n)
            off = off + (tails[i] & sz)

    stage(0, True)
    st0 = starts[0]

    def zero_block(k, carry):
        base = k * STEP

        @plsc.parallel_loop(0, STEP, step=L, unroll=UNROLL)
        def _zero(j):
            acc[pl.ds(base + j, L)] = zeros

        return carry

    lax.fori_loop(0, st0 // STEP, zero_block, None)

    for i in range(E):
        if i + 1 < E:
            stage(i + 1, True)
        if i == 0:
            stage(i, False)
            continue
        st = starts[i]
        cur = bufs[i % 2]
        sem = sems[i % 2]

        def wait_add_block(k, carry):
            base = k * CHUNK
            pltpu.make_async_copy(
                x_hbm.at[b, i, pl.ds(base, CHUNK)],
                cur.at[pl.ds(base, CHUNK)],
                sem,
            ).wait()

            @plsc.parallel_loop(0, CHUNK, step=L, unroll=UNROLL)
            def _add(j):
                plsc.addupdate(acc.at[pl.ds(st + base + j, L)],
                               cur[pl.ds(base + j, L)])

            return carry

        lax.fori_loop(0, nfulls[i], wait_add_block, None)

        tbase = nfulls[i] * CHUNK
        off = tbase
        for sz in TAILS:
            @pl.when((tails[i] & sz) != 0)
            def _piece(o=pl.multiple_of(off, STEP), n=sz):
                pltpu.make_async_copy(
                    x_hbm.at[b, i, pl.ds(o, n)],
                    cur.at[pl.ds(o, n)],
                    sem,
                ).wait()
            off = off + (tails[i] & sz)

        @plsc.parallel_loop(0, tails[i], step=L, unroll=UNROLL)
        def _tail_add(j):
            plsc.addupdate(acc.at[pl.ds(st + tbase + j, L)],
                           cur[pl.ds(tbase + j, L)])

    pltpu.sync_copy(acc, out_hbm.at[b, 0])


_mesh = plsc.VectorSubcoreMesh(core_axis_name="c", subcore_axis_name="s")

_summarize = functools.partial(
    pl.kernel,
    mesh=_mesh,
    out_type=jax.ShapeDtypeStruct((B, 1, S), jnp.float32),
    scratch_types=[
        pltpu.VMEM((E,), jnp.int32),
        pltpu.VMEM((S,), jnp.float32),
        pltpu.VMEM((S,), jnp.float32),
        pltpu.VMEM((S,), jnp.float32),
        pltpu.SemaphoreType.DMA,
        pltpu.SemaphoreType.DMA,
    ],
)(_worker)


def kernel(x, indices):
    return _summarize(x, indices.astype(jnp.int32))


# speedup vs baseline: 2.4558x; 1.0769x over previous
import functools

import jax
import jax.numpy as jnp
from jax import lax
from jax.experimental import pallas as pl
from jax.experimental.pallas import tpu as pltpu
from jax.experimental.pallas import tpu_sc as plsc

S = 32768
E = 16
B = 32
STEP = 256
L = 16
CHUNK = 2048
UNROLL = 8
ACC = S + CHUNK - STEP


def _worker(x_hbm, idx_hbm, out_hbm, idx_v, acc, buf_a, buf_b, sem_a, sem_b):
    c = lax.axis_index("c")
    s = lax.axis_index("s")
    b = s * 2 + c

    pltpu.sync_copy(idx_hbm.at[b], idx_v)
    vec = idx_v[...]

    zeros = jnp.zeros((L,), jnp.float32)

    bufs = (buf_a, buf_b)
    sems = (sem_a, sem_b)
    starts = [vec[i] * STEP for i in range(E)]
    nblocks = [(S - starts[i] + CHUNK - 1) // CHUNK for i in range(E)]

    def stage(i, is_start):
        if i == 0:
            dst = lambda k: acc.at[pl.ds(starts[0] + k * CHUNK, CHUNK)]
        else:
            buf = bufs[i % 2]
            dst = lambda k: buf.at[pl.ds(k * CHUNK, CHUNK)]
        sem = sems[i % 2]

        def body(k, carry):
            copy = pltpu.make_async_copy(
                x_hbm.at[b, i, pl.ds(k * CHUNK, CHUNK)],
                dst(k),
                sem,
            )
            if is_start:
                copy.start()
            else:
                copy.wait()
            return carry

        lax.fori_loop(0, nblocks[i], body, None)

    stage(0, True)
    st0 = starts[0]

    def zero_block(k, carry):
        base = k * STEP

        @plsc.parallel_loop(0, STEP, step=L, unroll=UNROLL)
        def _zero(j):
            acc[pl.ds(base + j, L)] = zeros

        return carry

    lax.fori_loop(0, st0 // STEP, zero_block, None)

    for i in range(E):
        if i + 1 < E:
            stage(i + 1, True)
        if i == 0:
            stage(i, False)
            continue
        st = starts[i]
        cur = bufs[i % 2]
        sem = sems[i % 2]

        def wait_add_block(k, carry):
            base = k * CHUNK
            pltpu.make_async_copy(
                x_hbm.at[b, i, pl.ds(base, CHUNK)],
                cur.at[pl.ds(base, CHUNK)],
                sem,
            ).wait()

            @plsc.parallel_loop(0, CHUNK, step=L, unroll=UNROLL)
            def _add(j):
                plsc.addupdate(acc.at[pl.ds(st + base + j, L)],
                               cur[pl.ds(base + j, L)])

            return carry

        lax.fori_loop(0, nblocks[i], wait_add_block, None)

    pltpu.sync_copy(acc.at[pl.ds(0, S)], out_hbm.at[b, 0])


_mesh = plsc.VectorSubcoreMesh(core_axis_name="c", subcore_axis_name="s")

_summarize = functools.partial(
    pl.kernel,
    mesh=_mesh,
    out_type=jax.ShapeDtypeStruct((B, 1, S), jnp.float32),
    scratch_types=[
        pltpu.VMEM((E,), jnp.int32),
        pltpu.VMEM((ACC,), jnp.float32),
        pltpu.VMEM((S,), jnp.float32),
        pltpu.VMEM((S,), jnp.float32),
        pltpu.SemaphoreType.DMA,
        pltpu.SemaphoreType.DMA,
    ],
)(_worker)


def kernel(x, indices):
    return _summarize(x, indices.astype(jnp.int32))
